# baseline (speedup 1.0000x reference)
import functools

import jax
import jax.numpy as jnp
from jax.experimental import pallas as pl
from jax.experimental.pallas import tpu as pltpu

_K_TOP = 32


def _encode_kernel(x_ref, w_ref, b_ref, out_ref):
    acc = jax.lax.dot_general(
        x_ref[...], w_ref[...],
        dimension_numbers=(((1,), (1,)), ((), ())),
        preferred_element_type=jnp.float32,
        precision=jax.lax.Precision.DEFAULT,
    )
    out_ref[...] = acc + b_ref[...]


def _threshold_kernel(p_ref, t_ref, lo_ref, hi_ref, cl_ref, ch_ref,
                      *, k, iters, interp_iters):
    P = p_ref[...]
    kf = jnp.float32(k)
    rows = P.shape[0]

    def count(t):
        return jnp.sum((P >= t).astype(jnp.float32), axis=1, keepdims=True)

    rmax = jnp.max(P, axis=1, keepdims=True)
    rmin = jnp.min(P, axis=1, keepdims=True)
    lo_ref[...] = rmin
    hi_ref[...] = rmax
    cl_ref[...] = jnp.full_like(rmax, jnp.float32(P.shape[1]))
    ch_ref[...] = jnp.full_like(rmax, jnp.float32(1.0))

    def cond(st):
        i, ndone = st
        return jnp.logical_and(i < iters, ndone < rows)

    def body(st):
        i, _ = st
        lo = lo_ref[...]
        hi = hi_ref[...]
        llo = jnp.log(jnp.maximum(cl_ref[...], 1.0))
        lhi = jnp.log(jnp.maximum(ch_ref[...], 0.5))
        lk = jnp.log(kf)
        frac = (llo - lk) / jnp.maximum(llo - lhi, jnp.float32(1e-6))
        frac = jnp.clip(frac, 0.08, 0.92)
        frac = jnp.where(i < interp_iters, frac, jnp.float32(0.5))
        mid = lo + frac * (hi - lo)
        c = count(mid)
        ge = c >= kf
        le = c <= kf
        lo_ref[...] = jnp.where(ge, mid, lo)
        hi_ref[...] = jnp.where(le, mid, hi)
        cl_ref[...] = jnp.where(ge, c, cl_ref[...])
        ch_ref[...] = jnp.where(le, c, ch_ref[...])
        done = jnp.logical_or(c == kf,
                              jnp.logical_or(mid == lo, mid == hi))
        return i + 1, jnp.sum(done.astype(jnp.float32))

    jax.lax.while_loop(cond, body, (jnp.int32(0), jnp.float32(0.0)))
    t_ref[...] = lo_ref[...]


def _decode_kernel(p_ref, w_ref, t_ref, acts_ref, recon_ref, *, rd):
    j = pl.program_id(0)
    r = pl.program_id(1)
    tile = p_ref[...]
    t = t_ref[...]
    acts = jnp.where(tile >= t, jnp.maximum(tile, 0.0), 0.0)
    acts_ref[...] = acts
    contrib = jax.lax.dot_general(
        acts, w_ref[...],
        dimension_numbers=(((1,), (1,)), ((), ())),
        preferred_element_type=jnp.float32,
        precision=jax.lax.Precision.DEFAULT,
    )
    rs = pl.ds(r * rd, rd)

    @pl.when(j == 0)
    def _():
        recon_ref[rs, :] = contrib

    @pl.when(j > 0)
    def _():
        recon_ref[rs, :] = recon_ref[rs, :] + contrib


def kernel(x, W_enc, b_enc, W_dec):
    n, d = x.shape
    nf = W_enc.shape[0]
    f32 = jnp.float32

    fj = min(2048, nf)
    nj1 = nf // fj
    re = min(1024, n)
    nre = n // re
    b2 = b_enc.reshape(1, nf).astype(f32)
    pre = pl.pallas_call(
        _encode_kernel,
        grid=(nj1, nre),
        in_specs=[
            pl.BlockSpec((re, d), lambda j, r: (r, 0)),
            pl.BlockSpec((fj, d), lambda j, r: (j, 0)),
            pl.BlockSpec((1, fj), lambda j, r: (0, j)),
        ],
        out_specs=pl.BlockSpec((re, fj), lambda j, r: (r, j)),
        out_shape=jax.ShapeDtypeStruct((n, nf), f32),
    )(x.astype(f32), W_enc.astype(f32), b2)

    rt = min(128, n)
    nrt = n // rt
    thr = pl.pallas_call(
        functools.partial(_threshold_kernel, k=_K_TOP, iters=46,
                          interp_iters=14),
        grid=(nrt,),
        in_specs=[pl.BlockSpec((rt, nf), lambda r: (r, 0))],
        out_specs=pl.BlockSpec((rt, 1), lambda r: (r, 0)),
        out_shape=jax.ShapeDtypeStruct((n, 1), f32),
        scratch_shapes=[
            pltpu.VMEM((rt, 1), f32),
            pltpu.VMEM((rt, 1), f32),
            pltpu.VMEM((rt, 1), f32),
            pltpu.VMEM((rt, 1), f32),
        ],
    )(pre)

    rd = min(1024, n)
    nrd = n // rd
    fd = min(1024, nf)
    nj2 = nf // fd
    acts, recon = pl.pallas_call(
        functools.partial(_decode_kernel, rd=rd),
        grid=(nj2, nrd),
        in_specs=[
            pl.BlockSpec((rd, fd), lambda j, r: (r, j)),
            pl.BlockSpec((d, fd), lambda j, r: (0, j)),
            pl.BlockSpec((rd, 1), lambda j, r: (r, 0)),
        ],
        out_specs=[
            pl.BlockSpec((rd, fd), lambda j, r: (r, j)),
            pl.BlockSpec((n, d), lambda j, r: (0, 0)),
        ],
        out_shape=[
            jax.ShapeDtypeStruct((n, nf), f32),
            jax.ShapeDtypeStruct((n, d), f32),
        ],
    )(pre, W_dec.astype(f32), thr)

    return recon, acts
